# baseline (speedup 1.0000x reference)
import functools

import jax
import jax.numpy as jnp
from jax import lax
from jax.experimental import pallas as pl
from jax.experimental.pallas import tpu as pltpu
from jax.experimental.pallas import tpu_sc as plsc

B = 16384
F = 26
V = 1000
D = 50
DP = 56
N = B * F

NC = 2
NS = 16
NW = NC * NS
NPW = N // NW
STREAM = 104
SC_CHUNK = 2 * STREAM
NG = NPW // SC_CHUNK


def _body(idx_hbm, tab_hbm, out_hbm, idx_v, rows_v, sem):
    wid = lax.axis_index("s") * NC + lax.axis_index("c")
    base = wid * NPW

    pltpu.sync_copy(idx_hbm.at[pl.ds(base, NPW)], idx_v)

    def g_body(g, carry):
        s0 = pl.multiple_of(g * SC_CHUNK, SC_CHUNK)
        cp0 = pltpu.make_async_copy(
            tab_hbm.at[idx_v.at[pl.ds(s0, STREAM)]],
            rows_v.at[pl.ds(0, STREAM)], sem)
        cp1 = pltpu.make_async_copy(
            tab_hbm.at[idx_v.at[pl.ds(s0 + STREAM, STREAM)]],
            rows_v.at[pl.ds(STREAM, STREAM)], sem)
        cp0.start()
        cp1.start()
        cp0.wait()
        cp1.wait()
        pltpu.sync_copy(rows_v, out_hbm.at[pl.ds(base + s0, SC_CHUNK)])
        return carry

    lax.fori_loop(0, NG, g_body, 0)


@functools.partial(
    pl.kernel,
    out_type=jax.ShapeDtypeStruct((N, DP), jnp.float32),
    mesh=plsc.VectorSubcoreMesh(core_axis_name="c", subcore_axis_name="s"),
    compiler_params=pltpu.CompilerParams(use_tc_tiling_on_sc=False),
    scratch_types=[
        pltpu.VMEM((NPW,), jnp.int32),
        pltpu.VMEM((SC_CHUNK, DP), jnp.float32),
        pltpu.SemaphoreType.DMA,
    ],
)
def _gather_kernel(idx_hbm, tab_hbm, out_hbm, idx_v, rows_v, sem):
    _body(idx_hbm, tab_hbm, out_hbm, idx_v, rows_v, sem)


def kernel(indices, tables):
    idx_flat = (indices.astype(jnp.int32)
                + jnp.arange(F, dtype=jnp.int32)[None, :] * V).reshape(N)
    tab_pad = jnp.pad(tables.reshape(F * V, D), ((0, 0), (0, DP - D)))
    out = _gather_kernel(idx_flat, tab_pad)
    out2 = out.reshape(B, F * DP)
    return jnp.concatenate(
        [out2[:, f * DP:f * DP + D] for f in range(F)], axis=1)


# speedup vs baseline: 1.4790x; 1.4790x over previous
import functools

import jax
import jax.numpy as jnp
from jax import lax
from jax.experimental import pallas as pl
from jax.experimental.pallas import tpu as pltpu
from jax.experimental.pallas import tpu_sc as plsc

B = 16384
F = 26
V = 1000
D = 50
DP = 56
N = B * F

NC = 2
NS = 16
NW = NC * NS
NPW = N // NW
STREAM = 104
SC_CHUNK = 2 * STREAM
NG = NPW // SC_CHUNK


def _body(idx_hbm, tab_hbm, out_hbm, idx_v, rows_v, sem):
    wid = lax.axis_index("s") * NC + lax.axis_index("c")
    base = wid * NPW

    pltpu.sync_copy(idx_hbm.at[pl.ds(base, NPW)], idx_v)

    def g_body(g, carry):
        s0 = pl.multiple_of(g * SC_CHUNK, SC_CHUNK)
        cp0 = pltpu.make_async_copy(
            tab_hbm.at[idx_v.at[pl.ds(s0, STREAM)]],
            rows_v.at[pl.ds(0, STREAM)], sem)
        cp1 = pltpu.make_async_copy(
            tab_hbm.at[idx_v.at[pl.ds(s0 + STREAM, STREAM)]],
            rows_v.at[pl.ds(STREAM, STREAM)], sem)
        cp0.start()
        cp1.start()
        cp0.wait()
        cp1.wait()
        pltpu.sync_copy(rows_v, out_hbm.at[pl.ds(base + s0, SC_CHUNK)])
        return carry

    lax.fori_loop(0, NG, g_body, 0)


@functools.partial(
    pl.kernel,
    out_type=jax.ShapeDtypeStruct((N, DP), jnp.float32),
    mesh=plsc.VectorSubcoreMesh(core_axis_name="c", subcore_axis_name="s"),
    compiler_params=pltpu.CompilerParams(use_tc_tiling_on_sc=False),
    scratch_types=[
        pltpu.VMEM((NPW,), jnp.int32),
        pltpu.VMEM((SC_CHUNK, DP), jnp.float32),
        pltpu.SemaphoreType.DMA,
    ],
)
def _gather_kernel(idx_hbm, tab_hbm, out_hbm, idx_v, rows_v, sem):
    _body(idx_hbm, tab_hbm, out_hbm, idx_v, rows_v, sem)


def kernel(indices, tables):
    idx_flat = (indices.astype(jnp.int32)
                + jnp.arange(F, dtype=jnp.int32)[None, :] * V).reshape(N)
    tab_pad = jnp.pad(tables.reshape(F * V, D), ((0, 0), (0, DP - D)))
    out = _gather_kernel(idx_flat, tab_pad)
    out2 = out.reshape(B, F * DP)
    cols = (jnp.arange(F * D, dtype=jnp.int32) // D) * DP + (
        jnp.arange(F * D, dtype=jnp.int32) % D)
    return jnp.take(out2, cols, axis=1, mode="clip")


# speedup vs baseline: 1.6176x; 1.0937x over previous
import functools

import jax
import jax.numpy as jnp
from jax import lax
from jax.experimental import pallas as pl
from jax.experimental.pallas import tpu as pltpu
from jax.experimental.pallas import tpu_sc as plsc

B = 16384
F = 26
V = 1000
D = 50
DP = 56
N = B * F

NC = 2
NS = 16
NW = NC * NS
NPW = N // NW
STREAM = 104
SC_CHUNK = 2 * STREAM
NG = NPW // SC_CHUNK


def _body(idx_hbm, tab_hbm, out_hbm, idx_v, rows0, rows1, sem0, sem1):
    wid = lax.axis_index("s") * NC + lax.axis_index("c")
    base = wid * NPW

    pltpu.sync_copy(idx_hbm.at[pl.ds(base, NPW)], idx_v)

    def copies(g, rows, sem):
        s0 = pl.multiple_of(g * SC_CHUNK, SC_CHUNK)
        cp0 = pltpu.make_async_copy(
            tab_hbm.at[idx_v.at[pl.ds(s0, STREAM)]],
            rows.at[pl.ds(0, STREAM)], sem)
        cp1 = pltpu.make_async_copy(
            tab_hbm.at[idx_v.at[pl.ds(s0 + STREAM, STREAM)]],
            rows.at[pl.ds(STREAM, STREAM)], sem)
        return cp0, cp1

    def fire(g, rows, sem):
        cp0, cp1 = copies(g, rows, sem)
        cp0.start()
        cp1.start()

    def drain(g, rows, sem):
        cp0, cp1 = copies(g, rows, sem)
        cp0.wait()
        cp1.wait()

    def store(g, rows):
        s0 = pl.multiple_of(g * SC_CHUNK, SC_CHUNK)
        pltpu.sync_copy(rows, out_hbm.at[pl.ds(base + s0, SC_CHUNK)])

    fire(0, rows0, sem0)

    def g_body(k, carry):
        g0 = pl.multiple_of(2 * k, 2)
        fire(g0 + 1, rows1, sem1)
        drain(g0, rows0, sem0)
        store(g0, rows0)

        @pl.when(g0 + 2 < NG)
        def _():
            fire(g0 + 2, rows0, sem0)

        drain(g0 + 1, rows1, sem1)
        store(g0 + 1, rows1)
        return carry

    lax.fori_loop(0, NG // 2, g_body, 0)


@functools.partial(
    pl.kernel,
    out_type=jax.ShapeDtypeStruct((N, DP), jnp.float32),
    mesh=plsc.VectorSubcoreMesh(core_axis_name="c", subcore_axis_name="s"),
    compiler_params=pltpu.CompilerParams(use_tc_tiling_on_sc=False),
    scratch_types=[
        pltpu.VMEM((NPW,), jnp.int32),
        pltpu.VMEM((SC_CHUNK, DP), jnp.float32),
        pltpu.VMEM((SC_CHUNK, DP), jnp.float32),
        pltpu.SemaphoreType.DMA,
        pltpu.SemaphoreType.DMA,
    ],
)
def _gather_kernel(idx_hbm, tab_hbm, out_hbm, idx_v, rows0, rows1, sem0, sem1):
    _body(idx_hbm, tab_hbm, out_hbm, idx_v, rows0, rows1, sem0, sem1)


def kernel(indices, tables):
    idx_flat = (indices.astype(jnp.int32)
                + jnp.arange(F, dtype=jnp.int32)[None, :] * V).reshape(N)
    tab_pad = jnp.pad(tables.reshape(F * V, D), ((0, 0), (0, DP - D)))
    out = _gather_kernel(idx_flat, tab_pad)
    out2 = out.reshape(B, F * DP)
    cols = (jnp.arange(F * D, dtype=jnp.int32) // D) * DP + (
        jnp.arange(F * D, dtype=jnp.int32) % D)
    return jnp.take(out2, cols, axis=1, mode="clip")


# speedup vs baseline: 1.6339x; 1.0101x over previous
import functools

import jax
import jax.numpy as jnp
from jax import lax
from jax.experimental import pallas as pl
from jax.experimental.pallas import tpu as pltpu
from jax.experimental.pallas import tpu_sc as plsc

B = 16384
F = 26
V = 1000
D = 50
DP = 56
N = B * F

NC = 2
NS = 16
NW = NC * NS
NPW = N // NW
STREAM = 104
NSTREAM = 4
SC_CHUNK = NSTREAM * STREAM
NG = NPW // SC_CHUNK


def _body(idx_hbm, tab_hbm, out_hbm, idx_v, rows0, rows1, sem0, sem1):
    wid = lax.axis_index("s") * NC + lax.axis_index("c")
    base = wid * NPW

    pltpu.sync_copy(idx_hbm.at[pl.ds(base, NPW)], idx_v)

    def copies(g, rows, sem):
        s0 = pl.multiple_of(g * SC_CHUNK, SC_CHUNK)
        return [
            pltpu.make_async_copy(
                tab_hbm.at[idx_v.at[pl.ds(s0 + j * STREAM, STREAM)]],
                rows.at[pl.ds(j * STREAM, STREAM)], sem)
            for j in range(NSTREAM)
        ]

    def fire(g, rows, sem):
        for cp in copies(g, rows, sem):
            cp.start()

    def drain(g, rows, sem):
        for cp in copies(g, rows, sem):
            cp.wait()

    def store(g, rows):
        s0 = pl.multiple_of(g * SC_CHUNK, SC_CHUNK)
        pltpu.sync_copy(rows, out_hbm.at[pl.ds(base + s0, SC_CHUNK)])

    fire(0, rows0, sem0)

    def g_body(k, carry):
        g0 = pl.multiple_of(2 * k, 2)
        fire(g0 + 1, rows1, sem1)
        drain(g0, rows0, sem0)
        store(g0, rows0)

        @pl.when(g0 + 2 < NG)
        def _():
            fire(g0 + 2, rows0, sem0)

        drain(g0 + 1, rows1, sem1)
        store(g0 + 1, rows1)
        return carry

    lax.fori_loop(0, NG // 2, g_body, 0)


@functools.partial(
    pl.kernel,
    out_type=jax.ShapeDtypeStruct((N, DP), jnp.float32),
    mesh=plsc.VectorSubcoreMesh(core_axis_name="c", subcore_axis_name="s"),
    compiler_params=pltpu.CompilerParams(use_tc_tiling_on_sc=False),
    scratch_types=[
        pltpu.VMEM((NPW,), jnp.int32),
        pltpu.VMEM((SC_CHUNK, DP), jnp.float32),
        pltpu.VMEM((SC_CHUNK, DP), jnp.float32),
        pltpu.SemaphoreType.DMA,
        pltpu.SemaphoreType.DMA,
    ],
)
def _gather_kernel(idx_hbm, tab_hbm, out_hbm, idx_v, rows0, rows1, sem0, sem1):
    _body(idx_hbm, tab_hbm, out_hbm, idx_v, rows0, rows1, sem0, sem1)


def kernel(indices, tables):
    idx_flat = (indices.astype(jnp.int32)
                + jnp.arange(F, dtype=jnp.int32)[None, :] * V).reshape(N)
    tab_pad = jnp.pad(tables.reshape(F * V, D), ((0, 0), (0, DP - D)))
    out = _gather_kernel(idx_flat, tab_pad)
    out2 = out.reshape(B, F * DP)
    cols = (jnp.arange(F * D, dtype=jnp.int32) // D) * DP + (
        jnp.arange(F * D, dtype=jnp.int32) % D)
    return jnp.take(out2, cols, axis=1, mode="clip")
